# baseline (speedup 1.0000x reference)
import functools

import jax
import jax.numpy as jnp
from jax import lax
from jax.experimental import pallas as pl
from jax.experimental.pallas import tpu as pltpu
from jax.experimental.pallas import tpu_sc as plsc

L = 16
C = 16
LN_EPS_ = 1e-5


def _emb_ln_sc(ids1d, pids1d, embed_tokens, embed_positions, ln_weight, ln_bias):
    N, = ids1d.shape
    V, D = embed_tokens.shape
    c = C
    info = plsc.get_sparse_core_info()
    NW = info.num_cores * info.num_subcores
    tok_per_w = N // NW
    nchunk = -(-tok_per_w // c)
    nchunk += nchunk % 2
    last_off = tok_per_w - c
    ngrp = D // L

    mesh = plsc.VectorSubcoreMesh(core_axis_name="c", subcore_axis_name="s")

    @functools.partial(
        pl.kernel,
        mesh=mesh,
        compiler_params=pltpu.CompilerParams(needs_layout_passes=False),
        out_type=jax.ShapeDtypeStruct((N, D), jnp.float32),
        scratch_types=[
            pltpu.VMEM((tok_per_w,), jnp.int32),
            pltpu.VMEM((tok_per_w,), jnp.int32),
            pltpu.VMEM((2, c, D), jnp.float32),
            pltpu.VMEM((2, c, D // 2), jnp.float32),
            pltpu.VMEM((2, c, D), jnp.float32),
            pltpu.VMEM((D,), jnp.float32),
            pltpu.VMEM((D,), jnp.float32),
            pltpu.SMEM((2, c), jnp.float32),
            pltpu.SemaphoreType.DMA,
            pltpu.SemaphoreType.DMA,
            pltpu.SemaphoreType.DMA,
            pltpu.SemaphoreType.DMA,
        ],
    )
    def k(ids_hbm, pids_hbm, tok_hbm, pos_hbm, w_hbm, b_hbm, out_hbm,
          idx_t, idx_p, tok, pos, obuf, w_v, b_v, stats_s,
          gsem0, gsem1, osem0, osem1):
        wid = lax.axis_index("s") * info.num_cores + lax.axis_index("c")
        tok0 = wid * tok_per_w
        pltpu.sync_copy(ids_hbm.at[pl.ds(tok0, tok_per_w)], idx_t)
        pltpu.sync_copy(pids_hbm.at[pl.ds(tok0, tok_per_w)], idx_p)
        pltpu.sync_copy(w_hbm, w_v)
        pltpu.sync_copy(b_hbm, b_v)

        gsems = (gsem0, gsem1)
        osems = (osem0, osem1)

        def off_of(j):
            return lax.min(j * c, last_off)

        def issue_gather(b, j):
            off = off_of(j)
            pltpu.async_copy(tok_hbm.at[idx_t.at[pl.ds(off, c)]],
                             tok.at[b], gsems[b])
            pltpu.async_copy(pos_hbm.at[idx_p.at[pl.ds(off, c)]],
                             pos.at[b], gsems[b])

        issue_gather(0, 0)
        issue_gather(1, 1)

        def do_chunk(b, j):
            off = off_of(j)
            tok_b = tok.at[b]
            pos_b = pos.at[b]
            obuf_b = obuf.at[b]
            pltpu.make_async_copy(tok_hbm.at[idx_t.at[pl.ds(off, c)]],
                                  tok_b, gsems[b]).wait()
            pltpu.make_async_copy(pos_hbm.at[idx_p.at[pl.ds(off, c)]],
                                  pos_b, gsems[b]).wait()

            @plsc.parallel_loop(0, c, 1, unroll=2)
            def token(t):
                zero = jnp.zeros((L,), jnp.float32)
                half = ngrp // 2

                @plsc.parallel_loop(0, half, 1, unroll=8,
                                    carry=(zero, zero, zero, zero))
                def acc(i, carry):
                    s0, q0, s1, q1 = carry
                    pw = lax.bitcast_convert_type(pos_b[t, pl.ds(i * L, L)],
                                                  jnp.int32)
                    plo = lax.bitcast_convert_type(pw << 16, jnp.float32)
                    phi = lax.bitcast_convert_type(
                        pw & jnp.int32(-65536), jnp.float32)
                    sl = pl.ds(i * L, L)
                    v = tok_b[t, sl] + plo
                    tok_b[t, sl] = v
                    s0 = s0 + v
                    q0 = q0 + v * v
                    sl = pl.ds((i + half) * L, L)
                    v = tok_b[t, sl] + phi
                    tok_b[t, sl] = v
                    s1 = s1 + v
                    q1 = q1 + v * v
                    return (s0, q0, s1, q1)

                s0, q0, s1, q1 = acc
                mu = jnp.sum(s0 + s1) * (1.0 / D)
                var = jnp.sum(q0 + q1) * (1.0 / D) - mu * mu
                x = var + LN_EPS_
                ib = lax.bitcast_convert_type(x, jnp.int32)
                y = lax.bitcast_convert_type(
                    jnp.int32(0x5F3759DF) - (ib >> 1), jnp.float32)
                for _ in range(2):
                    y = y * (1.5 - 0.5 * x * y * y)
                stats_s[0, t] = y
                stats_s[1, t] = -mu * y

            @pl.when(j >= 2)
            def _():
                pltpu.make_async_copy(obuf_b, out_hbm.at[pl.ds(tok0, c)],
                                      osems[b]).wait()

            @plsc.parallel_loop(0, ngrp, 1, unroll=16)
            def colgrp(g):
                sl = pl.ds(g * L, L)
                wg = w_v[sl]
                bg = b_v[sl]
                for t in range(c):
                    y_t = stats_s[0, t]
                    c1_t = stats_s[1, t]
                    t1 = tok_b[t, sl] * y_t + c1_t
                    obuf_b[t, sl] = t1 * wg + bg
            pltpu.async_copy(obuf_b, out_hbm.at[pl.ds(tok0 + off, c)], osems[b])

            @pl.when(j + 2 < nchunk)
            def _():
                issue_gather(b, j + 2)

        def body(jj, _):
            do_chunk(0, 2 * jj)
            do_chunk(1, 2 * jj + 1)
            return 0

        lax.fori_loop(0, nchunk // 2, body, 0)
        pltpu.make_async_copy(obuf.at[0], out_hbm.at[pl.ds(tok0, c)],
                              osem0).wait()
        pltpu.make_async_copy(obuf.at[1], out_hbm.at[pl.ds(tok0, c)],
                              osem1).wait()

    return k(ids1d, pids1d, embed_tokens, embed_positions, ln_weight, ln_bias)


def kernel(input_ids, position_ids, embed_tokens, embed_positions, ln_weight, ln_bias):
    B, S = input_ids.shape
    V, D = embed_tokens.shape
    N = B * S
    ids1d = input_ids.T.reshape(N).astype(jnp.int32)
    pids1d = position_ids.T.reshape(N).astype(jnp.int32)
    pb = embed_positions.astype(jnp.bfloat16)
    pos_packed = jax.lax.bitcast_convert_type(
        jnp.stack([pb[:, :D // 2], pb[:, D // 2:]], axis=-1), jnp.float32)
    out = _emb_ln_sc(ids1d, pids1d, embed_tokens, pos_packed,
                     ln_weight, ln_bias)
    return out.reshape(S, B, D).transpose(1, 0, 2)


# speedup vs baseline: 1.1259x; 1.1259x over previous
import functools

import jax
import jax.numpy as jnp
from jax import lax
from jax.experimental import pallas as pl
from jax.experimental.pallas import tpu as pltpu
from jax.experimental.pallas import tpu_sc as plsc

L = 16
C = 16
LN_EPS_ = 1e-5


def _emb_ln_sc(ids1d, pids1d, embed_tokens, embed_positions, ln_weight, ln_bias):
    N, = ids1d.shape
    V, D = embed_tokens.shape
    c = C
    info = plsc.get_sparse_core_info()
    NW = info.num_cores * info.num_subcores
    tok_per_w = N // NW
    nchunk = -(-tok_per_w // c)
    nchunk += nchunk % 2
    last_off = tok_per_w - c
    ngrp = D // L

    mesh = plsc.VectorSubcoreMesh(core_axis_name="c", subcore_axis_name="s")

    @functools.partial(
        pl.kernel,
        mesh=mesh,
        compiler_params=pltpu.CompilerParams(needs_layout_passes=False),
        out_type=jax.ShapeDtypeStruct((N, D), jnp.float32),
        scratch_types=[
            pltpu.VMEM((tok_per_w,), jnp.int32),
            pltpu.VMEM((tok_per_w,), jnp.int32),
            pltpu.VMEM((2, c, D), jnp.float32),
            pltpu.VMEM((2, c, D // 2), jnp.float32),
            pltpu.VMEM((2, c, D), jnp.float32),
            pltpu.VMEM((D,), jnp.float32),
            pltpu.VMEM((D,), jnp.float32),
            pltpu.SMEM((2, c), jnp.float32),
            pltpu.SemaphoreType.DMA,
            pltpu.SemaphoreType.DMA,
            pltpu.SemaphoreType.DMA,
            pltpu.SemaphoreType.DMA,
        ],
    )
    def k(ids_hbm, pids_hbm, tok_hbm, pos_hbm, w_hbm, b_hbm, out_hbm,
          idx_t, idx_p, tok, pos, obuf, w_v, b_v, stats_s,
          gsem0, gsem1, osem0, osem1):
        wid = lax.axis_index("s") * info.num_cores + lax.axis_index("c")
        tok0 = wid * tok_per_w
        pltpu.sync_copy(ids_hbm.at[pl.ds(tok0, tok_per_w)], idx_t)
        pltpu.sync_copy(pids_hbm.at[pl.ds(tok0, tok_per_w)], idx_p)
        pltpu.sync_copy(w_hbm, w_v)
        pltpu.sync_copy(b_hbm, b_v)

        gsems = (gsem0, gsem1)
        osems = (osem0, osem1)

        def off_of(j):
            return lax.min(j * c, last_off)

        def issue_gather(b, j):
            off = off_of(j)
            pltpu.async_copy(tok_hbm.at[idx_t.at[pl.ds(off, c)]],
                             tok.at[b], gsems[b])
            pltpu.async_copy(pos_hbm.at[idx_p.at[pl.ds(off, c)]],
                             pos.at[b], gsems[b])

        issue_gather(0, 0)
        issue_gather(1, 1)

        def do_chunk(b, j):
            off = off_of(j)
            tok_b = tok.at[b]
            pos_b = pos.at[b]
            obuf_b = obuf.at[b]
            pltpu.make_async_copy(tok_hbm.at[idx_t.at[pl.ds(off, c)]],
                                  tok_b, gsems[b]).wait()
            pltpu.make_async_copy(pos_hbm.at[idx_p.at[pl.ds(off, c)]],
                                  pos_b, gsems[b]).wait()

            @plsc.parallel_loop(0, c, 1, unroll=2)
            def token(t):
                zero = jnp.zeros((L,), jnp.float32)
                half = ngrp // 2

                @plsc.parallel_loop(0, half, 1, unroll=16,
                                    carry=(zero, zero, zero, zero))
                def acc(i, carry):
                    s0, q0, s1, q1 = carry
                    pw = lax.bitcast_convert_type(pos_b[t, pl.ds(i * L, L)],
                                                  jnp.int32)
                    plo = lax.bitcast_convert_type(pw << 16, jnp.float32)
                    phi = lax.bitcast_convert_type(
                        pw & jnp.int32(-65536), jnp.float32)
                    sl = pl.ds(i * L, L)
                    v = tok_b[t, sl] + plo
                    tok_b[t, sl] = v
                    s0 = s0 + v
                    q0 = q0 + v * v
                    sl = pl.ds((i + half) * L, L)
                    v = tok_b[t, sl] + phi
                    tok_b[t, sl] = v
                    s1 = s1 + v
                    q1 = q1 + v * v
                    return (s0, q0, s1, q1)

                s0, q0, s1, q1 = acc
                mu = jnp.sum(s0 + s1) * (1.0 / D)
                var = jnp.sum(q0 + q1) * (1.0 / D) - mu * mu
                x = var + LN_EPS_
                ib = lax.bitcast_convert_type(x, jnp.int32)
                y = lax.bitcast_convert_type(
                    jnp.int32(0x5F3759DF) - (ib >> 1), jnp.float32)
                for _ in range(2):
                    y = y * (1.5 - 0.5 * x * y * y)
                stats_s[0, t] = y
                stats_s[1, t] = -mu * y

            @pl.when(j >= 2)
            def _():
                pltpu.make_async_copy(obuf_b, out_hbm.at[pl.ds(tok0, c)],
                                      osems[b]).wait()

            @plsc.parallel_loop(0, ngrp, 1, unroll=8)
            def colgrp(g):
                sl = pl.ds(g * L, L)
                wg = w_v[sl]
                bg = b_v[sl]
                for t in range(c):
                    y_t = stats_s[0, t]
                    c1_t = stats_s[1, t]
                    t1 = tok_b[t, sl] * y_t + c1_t
                    obuf_b[t, sl] = t1 * wg + bg
            pltpu.async_copy(obuf_b, out_hbm.at[pl.ds(tok0 + off, c)], osems[b])

            @pl.when(j + 2 < nchunk)
            def _():
                issue_gather(b, j + 2)

        def body(jj, _):
            do_chunk(0, 2 * jj)
            do_chunk(1, 2 * jj + 1)
            return 0

        lax.fori_loop(0, nchunk // 2, body, 0)
        pltpu.make_async_copy(obuf.at[0], out_hbm.at[pl.ds(tok0, c)],
                              osem0).wait()
        pltpu.make_async_copy(obuf.at[1], out_hbm.at[pl.ds(tok0, c)],
                              osem1).wait()

    return k(ids1d, pids1d, embed_tokens, embed_positions, ln_weight, ln_bias)


def kernel(input_ids, position_ids, embed_tokens, embed_positions, ln_weight, ln_bias):
    B, S = input_ids.shape
    V, D = embed_tokens.shape
    N = B * S
    ids1d = input_ids.T.reshape(N).astype(jnp.int32)
    pids1d = position_ids.T.reshape(N).astype(jnp.int32)
    pb = embed_positions.astype(jnp.bfloat16)
    pos_packed = jax.lax.bitcast_convert_type(
        jnp.stack([pb[:, :D // 2], pb[:, D // 2:]], axis=-1), jnp.float32)
    out = _emb_ln_sc(ids1d, pids1d, embed_tokens, pos_packed,
                     ln_weight, ln_bias)
    return out.reshape(S, B, D).transpose(1, 0, 2)


# speedup vs baseline: 1.2637x; 1.1223x over previous
import functools

import jax
import jax.numpy as jnp
from jax import lax
from jax.experimental import pallas as pl
from jax.experimental.pallas import tpu as pltpu
from jax.experimental.pallas import tpu_sc as plsc

L = 16
C = 16
LN_EPS_ = 1e-5


def _emb_ln_sc(ids1d, pids1d, embed_tokens, embed_positions, ln_weight, ln_bias):
    N, = ids1d.shape
    V, D = embed_tokens.shape
    c = C
    info = plsc.get_sparse_core_info()
    NW = info.num_cores * info.num_subcores
    tok_per_w = N // NW
    nchunk = -(-tok_per_w // c)
    nchunk += nchunk % 2
    last_off = tok_per_w - c
    ngrp = D // L

    mesh = plsc.VectorSubcoreMesh(core_axis_name="c", subcore_axis_name="s")

    @functools.partial(
        pl.kernel,
        mesh=mesh,
        compiler_params=pltpu.CompilerParams(needs_layout_passes=False),
        out_type=jax.ShapeDtypeStruct((N, D), jnp.float32),
        scratch_types=[
            pltpu.VMEM((tok_per_w,), jnp.int32),
            pltpu.VMEM((tok_per_w,), jnp.int32),
            pltpu.VMEM((2, c, D), jnp.float32),
            pltpu.VMEM((2, c, D // 2), jnp.float32),
            pltpu.VMEM((2, c, D), jnp.float32),
            pltpu.VMEM((D,), jnp.float32),
            pltpu.VMEM((D,), jnp.float32),
            pltpu.SMEM((2, c), jnp.float32),
            pltpu.SemaphoreType.DMA,
            pltpu.SemaphoreType.DMA,
            pltpu.SemaphoreType.DMA,
            pltpu.SemaphoreType.DMA,
        ],
    )
    def k(ids_hbm, pids_hbm, tok_hbm, pos_hbm, w_hbm, b_hbm, out_hbm,
          idx_t, idx_p, tok, pos, obuf, w_v, b_v, stats_s,
          gsem0, gsem1, osem0, osem1):
        wid = lax.axis_index("s") * info.num_cores + lax.axis_index("c")
        tok0 = wid * tok_per_w
        pltpu.sync_copy(ids_hbm.at[pl.ds(tok0, tok_per_w)], idx_t)
        pltpu.sync_copy(pids_hbm.at[pl.ds(tok0, tok_per_w)], idx_p)
        pltpu.sync_copy(w_hbm, w_v)
        pltpu.sync_copy(b_hbm, b_v)

        gsems = (gsem0, gsem1)
        osems = (osem0, osem1)

        def off_of(j):
            return lax.min(j * c, last_off)

        def issue_gather(b, j):
            off = off_of(j)
            pltpu.async_copy(tok_hbm.at[idx_t.at[pl.ds(off, c)]],
                             tok.at[b], gsems[b])
            pltpu.async_copy(pos_hbm.at[idx_p.at[pl.ds(off, c)]],
                             pos.at[b], gsems[b])

        issue_gather(0, 0)
        issue_gather(1, 1)

        def do_chunk(b, j):
            off = off_of(j)
            tok_b = tok.at[b]
            pos_b = pos.at[b]
            obuf_b = obuf.at[b]
            pltpu.make_async_copy(tok_hbm.at[idx_t.at[pl.ds(off, c)]],
                                  tok_b, gsems[b]).wait()
            pltpu.make_async_copy(pos_hbm.at[idx_p.at[pl.ds(off, c)]],
                                  pos_b, gsems[b]).wait()

            @plsc.parallel_loop(0, c, 1, unroll=4)
            def token(t):
                zero = jnp.zeros((L,), jnp.float32)
                half = ngrp // 2

                @plsc.parallel_loop(0, half, 1, unroll=8,
                                    carry=(zero, zero, zero, zero))
                def acc(i, carry):
                    s0, q0, s1, q1 = carry
                    pw = lax.bitcast_convert_type(pos_b[t, pl.ds(i * L, L)],
                                                  jnp.int32)
                    plo = lax.bitcast_convert_type(pw << 16, jnp.float32)
                    phi = lax.bitcast_convert_type(
                        pw & jnp.int32(-65536), jnp.float32)
                    sl = pl.ds(i * L, L)
                    v = tok_b[t, sl] + plo
                    tok_b[t, sl] = v
                    s0 = s0 + v
                    q0 = q0 + v * v
                    sl = pl.ds((i + half) * L, L)
                    v = tok_b[t, sl] + phi
                    tok_b[t, sl] = v
                    s1 = s1 + v
                    q1 = q1 + v * v
                    return (s0, q0, s1, q1)

                s0, q0, s1, q1 = acc
                mu = jnp.sum(s0 + s1) * (1.0 / D)
                var = jnp.sum(q0 + q1) * (1.0 / D) - mu * mu
                x = var + LN_EPS_
                ib = lax.bitcast_convert_type(x, jnp.int32)
                y = lax.bitcast_convert_type(
                    jnp.int32(0x5F3759DF) - (ib >> 1), jnp.float32)
                for _ in range(2):
                    y = y * (1.5 - 0.5 * x * y * y)
                stats_s[0, t] = y
                stats_s[1, t] = -mu * y

            @pl.when(j >= 2)
            def _():
                pltpu.make_async_copy(obuf_b, out_hbm.at[pl.ds(tok0, c)],
                                      osems[b]).wait()

            @plsc.parallel_loop(0, ngrp, 1, unroll=8)
            def colgrp(g):
                sl = pl.ds(g * L, L)
                wg = w_v[sl]
                bg = b_v[sl]
                for t in range(c):
                    y_t = stats_s[0, t]
                    c1_t = stats_s[1, t]
                    t1 = tok_b[t, sl] * y_t + c1_t
                    obuf_b[t, sl] = t1 * wg + bg
            pltpu.async_copy(obuf_b, out_hbm.at[pl.ds(tok0 + off, c)], osems[b])

            @pl.when(j + 2 < nchunk)
            def _():
                issue_gather(b, j + 2)

        def body(jj, _):
            do_chunk(0, 2 * jj)
            do_chunk(1, 2 * jj + 1)
            return 0

        lax.fori_loop(0, nchunk // 2, body, 0)
        pltpu.make_async_copy(obuf.at[0], out_hbm.at[pl.ds(tok0, c)],
                              osem0).wait()
        pltpu.make_async_copy(obuf.at[1], out_hbm.at[pl.ds(tok0, c)],
                              osem1).wait()

    return k(ids1d, pids1d, embed_tokens, embed_positions, ln_weight, ln_bias)


def kernel(input_ids, position_ids, embed_tokens, embed_positions, ln_weight, ln_bias):
    B, S = input_ids.shape
    V, D = embed_tokens.shape
    N = B * S
    ids1d = input_ids.T.reshape(N).astype(jnp.int32)
    pids1d = position_ids.T.reshape(N).astype(jnp.int32)
    pb = embed_positions.astype(jnp.bfloat16)
    pos_packed = jax.lax.bitcast_convert_type(
        jnp.stack([pb[:, :D // 2], pb[:, D // 2:]], axis=-1), jnp.float32)
    out = _emb_ln_sc(ids1d, pids1d, embed_tokens, pos_packed,
                     ln_weight, ln_bias)
    return out.reshape(S, B, D).transpose(1, 0, 2)
